# baseline (speedup 1.0000x reference)
import jax
import jax.numpy as jnp
from jax.experimental import pallas as pl
from jax.experimental.pallas import tpu as pltpu

N = 100000
D_IN = 128
D_OUT = 20
EPS = 1e-5
BLK = 10000
NB = N // BLK


def _col(v_ref):
    return v_ref[...].reshape(1, D_OUT).T


def _body(x_hbm, w_ref, b_ref, g_ref, be_ref, o_ref,
          h_ref, xbuf, s1, s2, bb, sb, sem0, sem1, sem2, sem3):
    sems = ((sem0, sem1), (sem2, sem3))
    HB = BLK // 2

    def x_copies(i):
        slot = i % 2
        return [
            pltpu.make_async_copy(
                x_hbm.at[pl.ds(i * BLK + k * HB, HB), :],
                xbuf.at[slot, pl.ds(k * HB, HB), :],
                sems[slot][k])
            for k in range(2)
        ]

    s1[...] = jnp.zeros_like(s1)
    s2[...] = jnp.zeros_like(s2)
    bb[...] = jnp.broadcast_to(_col(b_ref), (D_OUT, BLK))

    for cp in x_copies(0):
        cp.start()
    for i in range(NB):
        if i + 1 < NB:
            for cp in x_copies(i + 1):
                cp.start()
        for cp in x_copies(i):
            cp.wait()
        z = jax.lax.dot_general(
            w_ref[...], xbuf[i % 2],
            (((1,), (1,)), ((), ())),
            preferred_element_type=jnp.float32,
        )
        h = jnp.tanh(z + bb[...])
        h_ref[i] = h
        s1[...] += jnp.sum(h, axis=1, keepdims=True)
        s2[...] += jnp.sum(h * h, axis=1, keepdims=True)

    mean = s1[...] * (1.0 / N)
    var = s2[...] * (1.0 / N) - mean * mean
    inv = jax.lax.rsqrt(var + EPS) * _col(g_ref)
    shift = _col(be_ref) - mean * inv
    bb[...] = jnp.broadcast_to(inv, (D_OUT, BLK))
    sb[...] = jnp.broadcast_to(shift, (D_OUT, BLK))
    for j in range(NB):
        o_ref[:, j * BLK:(j + 1) * BLK] = h_ref[j] * bb[...] + sb[...]


def kernel(x, W, b, gamma, beta):
    yt = pl.pallas_call(
        _body,
        grid=(1,),
        in_specs=[
            pl.BlockSpec(memory_space=pl.ANY),
            pl.BlockSpec((D_OUT, D_IN), lambda i: (0, 0)),
            pl.BlockSpec((D_OUT,), lambda i: (0,)),
            pl.BlockSpec((D_OUT,), lambda i: (0,)),
            pl.BlockSpec((D_OUT,), lambda i: (0,)),
        ],
        out_specs=pl.BlockSpec((D_OUT, N), lambda i: (0, 0)),
        out_shape=jax.ShapeDtypeStruct((D_OUT, N), jnp.float32),
        scratch_shapes=[
            pltpu.VMEM((NB, D_OUT, BLK), jnp.float32),
            pltpu.VMEM((2, BLK, D_IN), jnp.float32),
            pltpu.VMEM((D_OUT, 1), jnp.float32),
            pltpu.VMEM((D_OUT, 1), jnp.float32),
            pltpu.VMEM((D_OUT, BLK), jnp.float32),
            pltpu.VMEM((D_OUT, BLK), jnp.float32),
            pltpu.SemaphoreType.DMA,
            pltpu.SemaphoreType.DMA,
            pltpu.SemaphoreType.DMA,
            pltpu.SemaphoreType.DMA,
        ],
    )(x, W, b, gamma, beta)
    return yt.T


# speedup vs baseline: 1.0050x; 1.0050x over previous
import jax
import jax.numpy as jnp
from jax.experimental import pallas as pl
from jax.experimental.pallas import tpu as pltpu

N = 100000
D_IN = 128
D_OUT = 20
EPS = 1e-5
BLK = 20000
NB = N // BLK


def _col(v_ref):
    return v_ref[...].reshape(1, D_OUT).T


def _body(x_hbm, w_ref, b_ref, g_ref, be_ref, o_ref,
          h_ref, xbuf, s1, s2, bb, sb, sem0, sem1, sem2, sem3):
    sems = ((sem0, sem1), (sem2, sem3))
    HB = BLK // 2

    def x_copies(i):
        slot = i % 2
        return [
            pltpu.make_async_copy(
                x_hbm.at[pl.ds(i * BLK + k * HB, HB), :],
                xbuf.at[slot, pl.ds(k * HB, HB), :],
                sems[slot][k])
            for k in range(2)
        ]

    s1[...] = jnp.zeros_like(s1)
    s2[...] = jnp.zeros_like(s2)
    bb[...] = jnp.broadcast_to(_col(b_ref), (D_OUT, BLK))

    for cp in x_copies(0):
        cp.start()
    for i in range(NB):
        if i + 1 < NB:
            for cp in x_copies(i + 1):
                cp.start()
        for cp in x_copies(i):
            cp.wait()
        z = jax.lax.dot_general(
            w_ref[...], xbuf[i % 2],
            (((1,), (1,)), ((), ())),
            preferred_element_type=jnp.float32,
        )
        h = jnp.tanh(z + bb[...])
        h_ref[i] = h
        s1[...] += jnp.sum(h, axis=1, keepdims=True)
        s2[...] += jnp.sum(h * h, axis=1, keepdims=True)

    mean = s1[...] * (1.0 / N)
    var = s2[...] * (1.0 / N) - mean * mean
    inv = jax.lax.rsqrt(var + EPS) * _col(g_ref)
    shift = _col(be_ref) - mean * inv
    bb[...] = jnp.broadcast_to(inv, (D_OUT, BLK))
    sb[...] = jnp.broadcast_to(shift, (D_OUT, BLK))
    for j in range(NB):
        o_ref[:, j * BLK:(j + 1) * BLK] = h_ref[j] * bb[...] + sb[...]


def kernel(x, W, b, gamma, beta):
    yt = pl.pallas_call(
        _body,
        grid=(1,),
        in_specs=[
            pl.BlockSpec(memory_space=pl.ANY),
            pl.BlockSpec((D_OUT, D_IN), lambda i: (0, 0)),
            pl.BlockSpec((D_OUT,), lambda i: (0,)),
            pl.BlockSpec((D_OUT,), lambda i: (0,)),
            pl.BlockSpec((D_OUT,), lambda i: (0,)),
        ],
        out_specs=pl.BlockSpec((D_OUT, N), lambda i: (0, 0)),
        out_shape=jax.ShapeDtypeStruct((D_OUT, N), jnp.float32),
        scratch_shapes=[
            pltpu.VMEM((NB, D_OUT, BLK), jnp.float32),
            pltpu.VMEM((2, BLK, D_IN), jnp.float32),
            pltpu.VMEM((D_OUT, 1), jnp.float32),
            pltpu.VMEM((D_OUT, 1), jnp.float32),
            pltpu.VMEM((D_OUT, BLK), jnp.float32),
            pltpu.VMEM((D_OUT, BLK), jnp.float32),
            pltpu.SemaphoreType.DMA,
            pltpu.SemaphoreType.DMA,
            pltpu.SemaphoreType.DMA,
            pltpu.SemaphoreType.DMA,
        ],
    )(x, W, b, gamma, beta)
    return yt.T


# speedup vs baseline: 1.0086x; 1.0035x over previous
import jax
import jax.numpy as jnp
from jax.experimental import pallas as pl
from jax.experimental.pallas import tpu as pltpu

N = 100000
D_IN = 128
D_OUT = 20
EPS = 1e-5
BLK = 20000
NB = N // BLK


def _col(v_ref):
    return v_ref[...].reshape(1, D_OUT).T


def _body(x_hbm, w_ref, b_ref, g_ref, be_ref, o_ref,
          h_ref, xbuf, s1, s2, bb, sb, sem0, sem1, sem2, sem3):
    sems = ((sem0, sem1), (sem2, sem3))
    HB = BLK // 2

    CW = 12800

    def x_copies(i):
        slot = i % 2
        return [
            pltpu.make_async_copy(
                x_hbm.at[pl.ds(i * BLK + k * HB, HB), :],
                xbuf.at[slot, pl.ds(k * HB, HB), :],
                sems[slot][k])
            for k in range(2)
        ]

    s1[...] = jnp.zeros_like(s1)
    s2[...] = jnp.zeros_like(s2)
    bb[...] = jnp.broadcast_to(_col(b_ref), (D_OUT, BLK))

    for cp in x_copies(0):
        cp.start()
    for i in range(NB):
        if i + 1 < NB:
            for cp in x_copies(i + 1):
                cp.start()
        for cp in x_copies(i):
            cp.wait()
        z = jax.lax.dot_general(
            w_ref[...], xbuf[i % 2],
            (((1,), (1,)), ((), ())),
            preferred_element_type=jnp.float32,
        )
        h = jnp.tanh(z + bb[...])
        h_ref[:, i * BLK:(i + 1) * BLK] = h
        s1[...] += jnp.sum(h, axis=1, keepdims=True)
        s2[...] += jnp.sum(h * h, axis=1, keepdims=True)

    mean = s1[...] * (1.0 / N)
    var = s2[...] * (1.0 / N) - mean * mean
    inv = jax.lax.rsqrt(var + EPS) * _col(g_ref)
    shift = _col(be_ref) - mean * inv
    bb[...] = jnp.broadcast_to(inv, (D_OUT, BLK))
    sb[...] = jnp.broadcast_to(shift, (D_OUT, BLK))
    lo = 0
    while lo < N:
        w = min(CW, N - lo)
        o_ref[:, lo:lo + w] = h_ref[:, lo:lo + w] * bb[:, :w] + sb[:, :w]
        lo += w


def kernel(x, W, b, gamma, beta):
    yt = pl.pallas_call(
        _body,
        grid=(1,),
        in_specs=[
            pl.BlockSpec(memory_space=pl.ANY),
            pl.BlockSpec((D_OUT, D_IN), lambda i: (0, 0)),
            pl.BlockSpec((D_OUT,), lambda i: (0,)),
            pl.BlockSpec((D_OUT,), lambda i: (0,)),
            pl.BlockSpec((D_OUT,), lambda i: (0,)),
        ],
        out_specs=pl.BlockSpec((D_OUT, N), lambda i: (0, 0)),
        out_shape=jax.ShapeDtypeStruct((D_OUT, N), jnp.float32),
        scratch_shapes=[
            pltpu.VMEM((D_OUT, N), jnp.float32),
            pltpu.VMEM((2, BLK, D_IN), jnp.float32),
            pltpu.VMEM((D_OUT, 1), jnp.float32),
            pltpu.VMEM((D_OUT, 1), jnp.float32),
            pltpu.VMEM((D_OUT, BLK), jnp.float32),
            pltpu.VMEM((D_OUT, BLK), jnp.float32),
            pltpu.SemaphoreType.DMA,
            pltpu.SemaphoreType.DMA,
            pltpu.SemaphoreType.DMA,
            pltpu.SemaphoreType.DMA,
        ],
    )(x, W, b, gamma, beta)
    return yt.T
